# baseline (speedup 1.0000x reference)
import functools

import jax
import jax.numpy as jnp
from jax import lax
from jax.experimental import pallas as pl
from jax.experimental.pallas import tpu as pltpu
from jax.experimental.pallas import tpu_sc as plsc

E_DIM = 512
HALF = E_DIM // 2
H_DIM = 8
DH = E_DIM // H_DIM
TOPK = 5
BLK = 8192
SUB = BLK // 8


def _flash_kernel(q_ref, wq_ref, bq_ref, wk_ref, bank_ref, w_ref,
                  ck_s, m_s, l_s, w_s):
    i = pl.program_id(0)

    @pl.when(i == 0)
    def _():
        qp = lax.dot_general(q_ref[...], wq_ref[...], (((1,), (1,)), ((), ())),
                             preferred_element_type=jnp.float32, precision=lax.Precision.HIGHEST) + bq_ref[...]
        scale = 1.0 / (DH ** 0.5)
        for h in range(H_DIM):
            qph = qp[:, h * DH:(h + 1) * DH]
            wkh = wk_ref[h * DH:(h + 1) * DH, :]
            ck_s[h:h + 1, :] = lax.dot_general(
                qph, wkh, (((1,), (0,)), ((), ())),
                preferred_element_type=jnp.float32, precision=lax.Precision.HIGHEST) * scale
        m_s[...] = jnp.full_like(m_s, -jnp.inf)
        l_s[...] = jnp.zeros_like(l_s)
        w_s[...] = jnp.zeros_like(w_s)

    blk = bank_ref[...]
    s = lax.dot_general(ck_s[...], blk, (((1,), (1,)), ((), ())),
                        preferred_element_type=jnp.float32)
    m_prev = m_s[...]
    m_new = jnp.maximum(m_prev, jnp.max(s, axis=1, keepdims=True))
    alpha = jnp.exp(m_prev - m_new)
    p = jnp.exp(s - m_new)
    l_s[...] = l_s[...] * alpha + jnp.sum(p, axis=1, keepdims=True)
    w_s[...] = w_s[...] * alpha + lax.dot_general(
        p, blk, (((1,), (0,)), ((), ())),
        preferred_element_type=jnp.float32)
    m_s[...] = m_new

    @pl.when(i == pl.num_programs(0) - 1)
    def _():
        w_ref[...] = w_s[...] / l_s[...]


def _sim_topk_kernel(w_ref, wv_ref, bv_ref, wo_ref, bo_ref, bank_ref,
                     vals_ref, idx_ref, on_s, sim_s):
    i = pl.program_id(0)

    @pl.when(i == 0)
    def _():
        parts = []
        for h in range(H_DIM):
            wh = w_ref[h:h + 1, :]
            wvh = wv_ref[h * DH:(h + 1) * DH, :]
            parts.append(lax.dot_general(wh, wvh, (((1,), (1,)), ((), ())),
                                         preferred_element_type=jnp.float32, precision=lax.Precision.HIGHEST))
        ctx = jnp.concatenate(parts, axis=1) + bv_ref[...]
        attn_out = lax.dot_general(ctx, wo_ref[...], (((1,), (1,)), ((), ())),
                                   preferred_element_type=jnp.float32, precision=lax.Precision.HIGHEST) + bo_ref[...]
        n = jnp.sqrt(jnp.sum(attn_out * attn_out, axis=1, keepdims=True))
        on_s[...] = attn_out / jnp.maximum(n, 1e-8)

    on = on_s[...]
    ones = jnp.ones((1, E_DIM), dtype=jnp.float32)
    for j in range(8):
        blkj = bank_ref[j * SUB:(j + 1) * SUB, :]
        num = lax.dot_general(on, blkj, (((1,), (1,)), ((), ())),
                              preferred_element_type=jnp.float32, precision=lax.Precision.HIGHEST)
        nsq = lax.dot_general(ones, blkj * blkj, (((1,), (1,)), ((), ())),
                              preferred_element_type=jnp.float32, precision=lax.Precision.HIGHEST)
        sim_s[pl.ds(8 * i + j, 1), :] = num / jnp.maximum(jnp.sqrt(nsq), 1e-8)

    @pl.when(i == pl.num_programs(0) - 1)
    def _():
        s = sim_s[...]
        r_dim, c_dim = s.shape
        riota = lax.broadcasted_iota(jnp.int32, s.shape, 0)
        ciota = lax.broadcasted_iota(jnp.int32, s.shape, 1)
        fiota = riota * c_dim + ciota
        col8 = lax.broadcasted_iota(jnp.int32, (1, 8), 1)
        vals = jnp.zeros((1, 8), dtype=jnp.float32)
        idxs = jnp.zeros((1, 8), dtype=jnp.int32)
        big = jnp.int32(r_dim * c_dim)
        for t in range(TOPK):
            v = jnp.max(jnp.max(s, axis=0, keepdims=True), axis=1,
                        keepdims=True)
            cand = jnp.where(s == v, fiota, big)
            ix = jnp.min(jnp.min(cand, axis=0, keepdims=True), axis=1,
                         keepdims=True)
            vals = jnp.where(col8 == t, v, vals)
            idxs = jnp.where(col8 == t, ix, idxs)
            s = jnp.where(fiota == ix, -jnp.inf, s)
        vals_ref[...] = vals
        idx_ref[...] = idxs


def _sc_gather_body(idx_hbm, bank_hbm, out_hbm, idx_v, rows_v, sem):
    wid = lax.axis_index("s") * 2 + lax.axis_index("c")

    @pl.when(wid == 0)
    def _():
        pltpu.sync_copy(idx_hbm, idx_v)
        pltpu.async_copy(bank_hbm.at[idx_v], rows_v, sem).wait()
        pltpu.sync_copy(rows_v, out_hbm)


def kernel(query, memory_bank, Wq, Wk, Wv, bq, bk, bv, Wo, bo, top_k):
    del bk
    M = memory_bank.shape[0]
    nblk = M // BLK
    f32 = jnp.float32

    q2 = query.reshape(1, E_DIM)
    bq2 = bq.reshape(1, E_DIM)
    bv2 = bv.reshape(1, E_DIM)
    bo2 = bo.reshape(1, E_DIM)

    w = pl.pallas_call(
        _flash_kernel,
        grid=(nblk,),
        in_specs=[
            pl.BlockSpec((1, E_DIM), lambda i: (0, 0)),
            pl.BlockSpec((E_DIM, E_DIM), lambda i: (0, 0)),
            pl.BlockSpec((1, E_DIM), lambda i: (0, 0)),
            pl.BlockSpec((E_DIM, E_DIM), lambda i: (0, 0)),
            pl.BlockSpec((BLK, E_DIM), lambda i: (i, 0)),
        ],
        out_specs=pl.BlockSpec((H_DIM, E_DIM), lambda i: (0, 0)),
        out_shape=jax.ShapeDtypeStruct((H_DIM, E_DIM), f32),
        scratch_shapes=[
            pltpu.VMEM((H_DIM, E_DIM), f32),
            pltpu.VMEM((H_DIM, 1), f32),
            pltpu.VMEM((H_DIM, 1), f32),
            pltpu.VMEM((H_DIM, E_DIM), f32),
        ],
    )(q2, Wq, bq2, Wk, memory_bank)

    vals8, idx8 = pl.pallas_call(
        _sim_topk_kernel,
        grid=(nblk,),
        in_specs=[
            pl.BlockSpec((H_DIM, E_DIM), lambda i: (0, 0)),
            pl.BlockSpec((E_DIM, E_DIM), lambda i: (0, 0)),
            pl.BlockSpec((1, E_DIM), lambda i: (0, 0)),
            pl.BlockSpec((E_DIM, E_DIM), lambda i: (0, 0)),
            pl.BlockSpec((1, E_DIM), lambda i: (0, 0)),
            pl.BlockSpec((BLK, E_DIM), lambda i: (i, 0)),
        ],
        out_specs=(
            pl.BlockSpec((1, 8), lambda i: (0, 0)),
            pl.BlockSpec((1, 8), lambda i: (0, 0)),
        ),
        out_shape=(
            jax.ShapeDtypeStruct((1, 8), f32),
            jax.ShapeDtypeStruct((1, 8), jnp.int32),
        ),
        scratch_shapes=[
            pltpu.VMEM((1, E_DIM), f32),
            pltpu.VMEM((M // SUB, SUB), f32),
        ],
    )(w, Wv, bv2, Wo, bo2, memory_bank)

    top_vals = vals8[0, :TOPK]
    top_idx = idx8[0, :TOPK]

    sc_gather = functools.partial(
        pl.kernel,
        mesh=plsc.VectorSubcoreMesh(core_axis_name="c", subcore_axis_name="s"),
        out_type=jax.ShapeDtypeStruct((8, E_DIM), f32),
        scratch_types=[
            pltpu.VMEM((8,), jnp.int32),
            pltpu.VMEM((8, E_DIM), f32),
            pltpu.SemaphoreType.DMA,
        ],
    )(_sc_gather_body)
    retrieved8 = sc_gather(idx8.reshape(8), memory_bank)

    return top_vals, top_idx, retrieved8[:TOPK]


# speedup vs baseline: 2.6570x; 2.6570x over previous
import functools

import jax
import jax.numpy as jnp
from jax import lax
from jax.experimental import pallas as pl
from jax.experimental.pallas import tpu as pltpu
from jax.experimental.pallas import tpu_sc as plsc

E_DIM = 512
HALF = E_DIM // 2
H_DIM = 8
DH = E_DIM // H_DIM
TOPK = 5
NCAND = 16
BLK = 8192
SUB = BLK // 8


def _flash_kernel(q_ref, wq_ref, bq_ref, wk_ref, bank_ref, w_ref,
                  ck_s, m_s, l_s, w_s):
    i = pl.program_id(0)

    @pl.when(i == 0)
    def _():
        qp = lax.dot_general(q_ref[...], wq_ref[...], (((1,), (1,)), ((), ())),
                             preferred_element_type=jnp.float32, precision=lax.Precision.HIGHEST) + bq_ref[...]
        scale = 1.0 / (DH ** 0.5)
        for h in range(H_DIM):
            qph = qp[:, h * DH:(h + 1) * DH]
            wkh = wk_ref[h * DH:(h + 1) * DH, :]
            ck_s[h:h + 1, :] = lax.dot_general(
                qph, wkh, (((1,), (0,)), ((), ())),
                preferred_element_type=jnp.float32, precision=lax.Precision.HIGHEST) * scale
        m_s[...] = jnp.full_like(m_s, -jnp.inf)
        l_s[...] = jnp.zeros_like(l_s)
        w_s[...] = jnp.zeros_like(w_s)

    blk = bank_ref[...]
    s = lax.dot_general(ck_s[...], blk, (((1,), (1,)), ((), ())),
                        preferred_element_type=jnp.float32)
    m_prev = m_s[...]
    m_new = jnp.maximum(m_prev, jnp.max(s, axis=1, keepdims=True))
    alpha = jnp.exp(m_prev - m_new)
    p = jnp.exp(s - m_new)
    l_s[...] = l_s[...] * alpha + jnp.sum(p, axis=1, keepdims=True)
    w_s[...] = w_s[...] * alpha + lax.dot_general(
        p, blk, (((1,), (0,)), ((), ())),
        preferred_element_type=jnp.float32)
    m_s[...] = m_new

    @pl.when(i == pl.num_programs(0) - 1)
    def _():
        w_ref[...] = w_s[...] / l_s[...]


def _sim_topk_kernel(w_ref, wv_ref, bv_ref, wo_ref, bo_ref, bank_ref,
                     idx_ref, on_ref, on_s, sim_s):
    i = pl.program_id(0)

    @pl.when(i == 0)
    def _():
        parts = []
        for h in range(H_DIM):
            wh = w_ref[h:h + 1, :]
            wvh = wv_ref[h * DH:(h + 1) * DH, :]
            parts.append(lax.dot_general(wh, wvh, (((1,), (1,)), ((), ())),
                                         preferred_element_type=jnp.float32, precision=lax.Precision.HIGHEST))
        ctx = jnp.concatenate(parts, axis=1) + bv_ref[...]
        attn_out = lax.dot_general(ctx, wo_ref[...], (((1,), (1,)), ((), ())),
                                   preferred_element_type=jnp.float32, precision=lax.Precision.HIGHEST) + bo_ref[...]
        n = jnp.sqrt(jnp.sum(attn_out * attn_out, axis=1, keepdims=True))
        on_s[...] = attn_out / jnp.maximum(n, 1e-8)
        on_ref[...] = on_s[...]

    on = on_s[...]
    ones = jnp.ones((1, E_DIM), dtype=jnp.float32)
    for j in range(8):
        blkj = bank_ref[j * SUB:(j + 1) * SUB, :]
        num = lax.dot_general(on, blkj, (((1,), (1,)), ((), ())),
                              preferred_element_type=jnp.float32)
        nsq = lax.dot_general(ones, blkj * blkj, (((1,), (1,)), ((), ())),
                              preferred_element_type=jnp.float32)
        sim_s[pl.ds(8 * i + j, 1), :] = num / jnp.maximum(jnp.sqrt(nsq), 1e-8)

    @pl.when(i == pl.num_programs(0) - 1)
    def _():
        s = sim_s[...]
        r_dim, c_dim = s.shape
        riota = lax.broadcasted_iota(jnp.int32, s.shape, 0)
        ciota = lax.broadcasted_iota(jnp.int32, s.shape, 1)
        fiota = riota * c_dim + ciota
        colc = lax.broadcasted_iota(jnp.int32, (1, NCAND), 1)
        idxs = jnp.zeros((1, NCAND), dtype=jnp.int32)
        big = jnp.int32(r_dim * c_dim)
        for t in range(NCAND):
            v = jnp.max(jnp.max(s, axis=0, keepdims=True), axis=1,
                        keepdims=True)
            cand = jnp.where(s == v, fiota, big)
            ix = jnp.min(jnp.min(cand, axis=0, keepdims=True), axis=1,
                         keepdims=True)
            idxs = jnp.where(colc == t, ix, idxs)
            s = jnp.where(fiota == ix, -jnp.inf, s)
        idx_ref[...] = idxs


def _sc_gather_body(idx_hbm, bank_hbm, out_hbm, idx_v, rows_v, sem):
    wid = lax.axis_index("s") * 2 + lax.axis_index("c")

    @pl.when(wid == 0)
    def _():
        pltpu.sync_copy(idx_hbm, idx_v)
        pltpu.async_copy(bank_hbm.at[idx_v], rows_v, sem).wait()
        pltpu.sync_copy(rows_v, out_hbm)


def _refine_kernel(rows_ref, on_ref, gidx_col_ref, gidx_row_ref,
                   vals_ref, idx_ref, retr_ref):
    rows = rows_ref[...]
    on = on_ref[...]
    s16 = jnp.sum(rows * on, axis=1, keepdims=True)
    nsq = jnp.sum(rows * rows, axis=1, keepdims=True)
    sim = s16 / jnp.maximum(jnp.sqrt(nsq), 1e-8)
    gcol = gidx_col_ref[...]
    grow = gidx_row_ref[...]
    col8 = lax.broadcasted_iota(jnp.int32, (1, 8), 1)
    vals = jnp.zeros((1, 8), dtype=jnp.float32)
    idxs = jnp.zeros((1, 8), dtype=jnp.int32)
    big = jnp.int32(2 ** 30)
    oh = []
    for t in range(TOPK):
        v = jnp.max(sim, axis=0, keepdims=True)
        hit = sim == v
        ix = jnp.min(jnp.where(hit, gcol, big), axis=0, keepdims=True)
        vals = jnp.where(col8 == t, v, vals)
        idxs = jnp.where(col8 == t, ix, idxs)
        oh.append((grow == ix).astype(jnp.float32))
        sim = jnp.where(gcol == ix, -jnp.inf, sim)
    onehot = jnp.concatenate(
        oh + [jnp.zeros((8 - TOPK, NCAND), dtype=jnp.float32)], axis=0)
    retr_ref[...] = lax.dot_general(
        onehot, rows, (((1,), (0,)), ((), ())),
        preferred_element_type=jnp.float32,
        precision=lax.Precision.HIGHEST)
    vals_ref[...] = vals
    idx_ref[...] = idxs


def kernel(query, memory_bank, Wq, Wk, Wv, bq, bk, bv, Wo, bo, top_k):
    del bk
    M = memory_bank.shape[0]
    nblk = M // BLK
    f32 = jnp.float32

    q2 = query.reshape(1, E_DIM)
    bq2 = bq.reshape(1, E_DIM)
    bv2 = bv.reshape(1, E_DIM)
    bo2 = bo.reshape(1, E_DIM)

    w = pl.pallas_call(
        _flash_kernel,
        grid=(nblk,),
        in_specs=[
            pl.BlockSpec((1, E_DIM), lambda i: (0, 0)),
            pl.BlockSpec((E_DIM, E_DIM), lambda i: (0, 0)),
            pl.BlockSpec((1, E_DIM), lambda i: (0, 0)),
            pl.BlockSpec((E_DIM, E_DIM), lambda i: (0, 0)),
            pl.BlockSpec((BLK, E_DIM), lambda i: (i, 0)),
        ],
        out_specs=pl.BlockSpec((H_DIM, E_DIM), lambda i: (0, 0)),
        out_shape=jax.ShapeDtypeStruct((H_DIM, E_DIM), f32),
        scratch_shapes=[
            pltpu.VMEM((H_DIM, E_DIM), f32),
            pltpu.VMEM((H_DIM, 1), f32),
            pltpu.VMEM((H_DIM, 1), f32),
            pltpu.VMEM((H_DIM, E_DIM), f32),
        ],
    )(q2, Wq, bq2, Wk, memory_bank)

    idx16, on_v = pl.pallas_call(
        _sim_topk_kernel,
        grid=(nblk,),
        in_specs=[
            pl.BlockSpec((H_DIM, E_DIM), lambda i: (0, 0)),
            pl.BlockSpec((E_DIM, E_DIM), lambda i: (0, 0)),
            pl.BlockSpec((1, E_DIM), lambda i: (0, 0)),
            pl.BlockSpec((E_DIM, E_DIM), lambda i: (0, 0)),
            pl.BlockSpec((1, E_DIM), lambda i: (0, 0)),
            pl.BlockSpec((BLK, E_DIM), lambda i: (i, 0)),
        ],
        out_specs=(
            pl.BlockSpec((1, NCAND), lambda i: (0, 0)),
            pl.BlockSpec((1, E_DIM), lambda i: (0, 0)),
        ),
        out_shape=(
            jax.ShapeDtypeStruct((1, NCAND), jnp.int32),
            jax.ShapeDtypeStruct((1, E_DIM), f32),
        ),
        scratch_shapes=[
            pltpu.VMEM((1, E_DIM), f32),
            pltpu.VMEM((M // SUB, SUB), f32),
        ],
    )(w, Wv, bv2, Wo, bo2, memory_bank)

    sc_gather = functools.partial(
        pl.kernel,
        mesh=plsc.VectorSubcoreMesh(core_axis_name="c", subcore_axis_name="s"),
        out_type=jax.ShapeDtypeStruct((NCAND, E_DIM), f32),
        scratch_types=[
            pltpu.VMEM((NCAND,), jnp.int32),
            pltpu.VMEM((NCAND, E_DIM), f32),
            pltpu.SemaphoreType.DMA,
        ],
    )(_sc_gather_body)
    rows16 = sc_gather(idx16.reshape(NCAND), memory_bank)

    vals8, idx8, retr8 = pl.pallas_call(
        _refine_kernel,
        out_shape=(
            jax.ShapeDtypeStruct((1, 8), f32),
            jax.ShapeDtypeStruct((1, 8), jnp.int32),
            jax.ShapeDtypeStruct((8, E_DIM), f32),
        ),
    )(rows16, on_v, idx16.reshape(NCAND, 1), idx16)

    return vals8[0, :TOPK], idx8[0, :TOPK], retr8[:TOPK]
